# baseline (speedup 1.0000x reference)
import jax
import jax.numpy as jnp
from jax.experimental import pallas as pl
from jax.experimental.pallas import tpu as pltpu

N = 2048
TOPK = 32
BLK = 256
F32 = jnp.float32


HI = jax.lax.Precision.HIGHEST


def _proj_norm_body(h_ref, w_ref, o_ref):
    p = jnp.dot(h_ref[...], w_ref[...], preferred_element_type=F32)
    pt = p.T
    s = jnp.sum(pt * pt, axis=0, keepdims=True)
    nrm = jnp.maximum(jnp.sqrt(s), 1e-12)
    o_ref[...] = (pt / nrm).T


def _proj_norm(h, w):
    m, f = h.shape
    u = w.shape[1]
    return pl.pallas_call(
        _proj_norm_body,
        grid=(m // BLK,),
        in_specs=[
            pl.BlockSpec((BLK, f), lambda i: (i, 0)),
            pl.BlockSpec((f, u), lambda i: (0, 0)),
        ],
        out_specs=pl.BlockSpec((BLK, u), lambda i: (i, 0)),
        out_shape=jax.ShapeDtypeStruct((m, u), F32),
    )(h, w)


def _sim_topk_body(hd_ref, ht_ref, bprev_ref, bout_ref, rs_ref, s_scr, w_scr):
    x = jax.lax.dot_general(hd_ref[...], ht_ref[...],
                            (((1,), (1,)), ((), ())),
                            preferred_element_type=F32)
    s = jax.nn.sigmoid(x)
    s_scr[...] = s
    w_scr[...] = s
    for _ in range(TOPK - 1):
        m = jnp.max(w_scr[...], axis=1, keepdims=True)
        w_scr[...] = jnp.where(w_scr[...] >= m, -1.0, w_scr[...])
    t = jnp.max(w_scr[...], axis=1, keepdims=True)
    s = s_scr[...]
    bnew = bprev_ref[...] + jnp.where(s >= t, s, 0.0)
    bout_ref[...] = bnew
    rs_ref[...] = jnp.sum(bnew, axis=1, keepdims=True)


def _sim_topk(hdp, htp, bprev):
    u = hdp.shape[1]
    return pl.pallas_call(
        _sim_topk_body,
        grid=(N // BLK,),
        in_specs=[
            pl.BlockSpec((BLK, u), lambda i: (i, 0)),
            pl.BlockSpec((N, u), lambda i: (0, 0)),
            pl.BlockSpec((BLK, N), lambda i: (i, 0)),
        ],
        out_specs=[
            pl.BlockSpec((BLK, N), lambda i: (i, 0)),
            pl.BlockSpec((BLK, 1), lambda i: (i, 0)),
        ],
        out_shape=[
            jax.ShapeDtypeStruct((N, N), F32),
            jax.ShapeDtypeStruct((N, 1), F32),
        ],
        scratch_shapes=[
            pltpu.VMEM((BLK, N), F32),
            pltpu.VMEM((BLK, N), F32),
        ],
    )(hdp, htp, bprev)


def _ptop_body(hd_ref, wg_ref, rs_ref, o_ref):
    dd = jax.lax.rsqrt(1.0 + rs_ref[...])
    o_ref[...] = dd * jnp.dot(hd_ref[...], wg_ref[...],
                              preferred_element_type=F32)


def _ptop(hd, wg, rs):
    f, u = wg.shape
    return pl.pallas_call(
        _ptop_body,
        grid=(N // BLK,),
        in_specs=[
            pl.BlockSpec((BLK, f), lambda i: (i, 0)),
            pl.BlockSpec((f, u), lambda i: (0, 0)),
            pl.BlockSpec((BLK, 1), lambda i: (i, 0)),
        ],
        out_specs=pl.BlockSpec((BLK, u), lambda i: (i, 0)),
        out_shape=jax.ShapeDtypeStruct((N, u), F32),
    )(hd, wg, rs)


def _bot_body(b_ref, ht_ref, wg_ref, ptop_ref, obot_ref, pbot_ref):
    b = b_ref[...]
    cs = jax.lax.dot_general(b, jnp.ones((N, 1), F32),
                             (((0,), (0,)), ((), ())),
                             preferred_element_type=F32)
    dt = jax.lax.rsqrt(1.0 + cs)
    mt = jnp.dot(ht_ref[...], wg_ref[...], preferred_element_type=F32)
    pbot = dt * mt
    pbot_ref[...] = pbot
    btp = jax.lax.dot_general(b, ptop_ref[...],
                              (((0,), (0,)), ((), ())),
                              preferred_element_type=F32)
    obot_ref[...] = jnp.maximum(dt * (pbot + btp), 0.0)


def _bot(bmat, ht, wg, ptop):
    f, u = wg.shape
    return pl.pallas_call(
        _bot_body,
        grid=(N // BLK,),
        in_specs=[
            pl.BlockSpec((N, BLK), lambda j: (0, j)),
            pl.BlockSpec((BLK, f), lambda j: (j, 0)),
            pl.BlockSpec((f, u), lambda j: (0, 0)),
            pl.BlockSpec((N, u), lambda j: (0, 0)),
        ],
        out_specs=[
            pl.BlockSpec((BLK, u), lambda j: (j, 0)),
            pl.BlockSpec((BLK, u), lambda j: (j, 0)),
        ],
        out_shape=[
            jax.ShapeDtypeStruct((N, u), F32),
            jax.ShapeDtypeStruct((N, u), F32),
        ],
    )(bmat, ht, wg, ptop)


def _top_body(b_ref, hd_ref, wg_ref, rs_ref, pbot_ref, otop_ref):
    dd = jax.lax.rsqrt(1.0 + rs_ref[...])
    md = jnp.dot(hd_ref[...], wg_ref[...], preferred_element_type=F32)
    acc = jnp.dot(b_ref[...], pbot_ref[...], preferred_element_type=F32)
    otop_ref[...] = jnp.maximum(dd * (dd * md + acc), 0.0)


def _top(bmat, hd, wg, rs, pbot):
    f, u = wg.shape
    return pl.pallas_call(
        _top_body,
        grid=(N // BLK,),
        in_specs=[
            pl.BlockSpec((BLK, N), lambda i: (i, 0)),
            pl.BlockSpec((BLK, f), lambda i: (i, 0)),
            pl.BlockSpec((f, u), lambda i: (0, 0)),
            pl.BlockSpec((BLK, 1), lambda i: (i, 0)),
            pl.BlockSpec((N, u), lambda i: (0, 0)),
        ],
        out_specs=pl.BlockSpec((BLK, u), lambda i: (i, 0)),
        out_shape=jax.ShapeDtypeStruct((N, u), F32),
    )(bmat, hd, wg, rs, pbot)


def _mm_body(h_ref, w_ref, o_ref):
    o_ref[...] = jnp.dot(h_ref[...], w_ref[...], preferred_element_type=F32)


def _mm(h, w):
    m, f = h.shape
    u = w.shape[1]
    return pl.pallas_call(
        _mm_body,
        grid=(m // BLK,),
        in_specs=[pl.BlockSpec((BLK, f), lambda i: (i, 0)),
                  pl.BlockSpec((f, u), lambda i: (0, 0))],
        out_specs=pl.BlockSpec((BLK, u), lambda i: (i, 0)),
        out_shape=jax.ShapeDtypeStruct((m, u), F32),
    )(h, w)


def _eye_seg(i_off, rows, cols):
    ri = jax.lax.broadcasted_iota(jnp.int32, (rows, cols), 0)
    ci = jax.lax.broadcasted_iota(jnp.int32, (rows, cols), 1)
    return jnp.where(ci == ri + i_off, 1.0, 0.0).astype(F32)


def _deg_top_body(b_ref, o_ref):
    i = pl.program_id(0)
    strip = jnp.concatenate([_eye_seg(i * BLK, BLK, N), b_ref[...]], axis=1)
    o_ref[...] = jnp.sum(strip.T, axis=0, keepdims=True)


def _deg_bot_body(b_ref, o_ref):
    i = pl.program_id(0)
    strip = jnp.concatenate([b_ref[...].T, _eye_seg(i * BLK, BLK, N)], axis=1)
    o_ref[...] = jnp.sum(strip.T, axis=0, keepdims=True)


def _degrees(bm):
    dtop = pl.pallas_call(
        _deg_top_body,
        grid=(N // BLK,),
        in_specs=[pl.BlockSpec((BLK, N), lambda i: (i, 0))],
        out_specs=pl.BlockSpec((1, BLK), lambda i: (0, i)),
        out_shape=jax.ShapeDtypeStruct((1, N), F32),
    )(bm)
    dbot = pl.pallas_call(
        _deg_bot_body,
        grid=(N // BLK,),
        in_specs=[pl.BlockSpec((N, BLK), lambda i: (0, i))],
        out_specs=pl.BlockSpec((1, BLK), lambda i: (0, i)),
        out_shape=jax.ShapeDtypeStruct((1, N), F32),
    )(bm)
    return jnp.concatenate([dtop, dbot], axis=1)


def _gcn_strip_body(is_top, b_ref, m_ref, dall_ref, dme_ref, o_ref):
    i = pl.program_id(0)
    if is_top:
        strip = jnp.concatenate([_eye_seg(i * BLK, BLK, N), b_ref[...]],
                                axis=1)
    else:
        strip = jnp.concatenate([b_ref[...].T, _eye_seg(i * BLK, BLK, N)],
                                axis=1)
    dall = jax.lax.rsqrt(dall_ref[...])
    drow = jax.lax.rsqrt(dme_ref[...].reshape(1, BLK)).T
    an = (drow * strip) * dall
    o_ref[...] = jnp.maximum(
        jnp.dot(an, m_ref[...], preferred_element_type=F32), 0.0)


def _gcn_exact(bm, m_full, dvec):
    d3 = dvec.reshape(2 * N // BLK, 1, BLK)
    u = m_full.shape[1]
    htop = pl.pallas_call(
        lambda *rs: _gcn_strip_body(True, *rs),
        grid=(N // BLK,),
        in_specs=[pl.BlockSpec((BLK, N), lambda i: (i, 0)),
                  pl.BlockSpec((2 * N, u), lambda i: (0, 0)),
                  pl.BlockSpec((1, 2 * N), lambda i: (0, 0)),
                  pl.BlockSpec((1, 1, BLK), lambda i: (i, 0, 0))],
        out_specs=pl.BlockSpec((BLK, u), lambda i: (i, 0)),
        out_shape=jax.ShapeDtypeStruct((N, u), F32),
    )(bm, m_full, dvec, d3)
    hbot = pl.pallas_call(
        lambda *rs: _gcn_strip_body(False, *rs),
        grid=(N // BLK,),
        in_specs=[pl.BlockSpec((N, BLK), lambda i: (0, i)),
                  pl.BlockSpec((2 * N, u), lambda i: (0, 0)),
                  pl.BlockSpec((1, 2 * N), lambda i: (0, 0)),
                  pl.BlockSpec((1, 1, BLK), lambda i: (i + N // BLK, 0, 0))],
        out_specs=pl.BlockSpec((BLK, u), lambda i: (i, 0)),
        out_shape=jax.ShapeDtypeStruct((N, u), F32),
    )(bm, m_full, dvec, d3)
    return htop, hbot


def _pred_body(hd_ref, ht_ref, o_ref):
    o_ref[...] = jax.lax.dot_general(hd_ref[...], ht_ref[...],
                                     (((1,), (1,)), ((), ())),
                                     preferred_element_type=F32)


def _pred(hd, ht):
    u = hd.shape[1]
    return pl.pallas_call(
        _pred_body,
        grid=(N // BLK,),
        in_specs=[
            pl.BlockSpec((BLK, u), lambda i: (i, 0)),
            pl.BlockSpec((N, u), lambda i: (0, 0)),
        ],
        out_specs=pl.BlockSpec((BLK, N), lambda i: (i, 0)),
        out_shape=jax.ShapeDtypeStruct((N, N), F32),
    )(hd, ht)


def kernel(H_d, H_t, A, W1_0, W2_0, Wg_0, W1_1, W2_1, Wg_1):
    bmat = A[:N, N:]

    hdp = _proj_norm(H_d, W1_0)
    htp = _proj_norm(H_t, W2_0)
    bmat, _ = _sim_topk(hdp, htp, bmat)
    m_full = jnp.concatenate([_mm(H_d, Wg_0), _mm(H_t, Wg_0)], axis=0)
    dvec = _degrees(bmat)
    H_d, H_t = _gcn_exact(bmat, m_full, dvec)

    hdp = _proj_norm(H_d, W1_1)
    htp = _proj_norm(H_t, W2_1)
    bmat, rs = _sim_topk(hdp, htp, bmat)
    ptop = _ptop(H_d, Wg_1, rs)
    ht_new, pbot = _bot(bmat, H_t, Wg_1, ptop)
    hd_new = _top(bmat, H_d, Wg_1, rs, pbot)
    H_d, H_t = hd_new, ht_new

    r_pred = _pred(H_d, H_t)
    return (r_pred, H_d, H_t)
